# Initial kernel scaffold; baseline (speedup 1.0000x reference)
import jax
import jax.numpy as jnp
from jax.experimental import pallas as pl


def kernel(feat1, label1, feat2, label2):
    raise NotImplementedError("write your pallas kernel here")


# speedup vs baseline: 7.2619x; 7.2619x over previous
import functools

import jax
import jax.numpy as jnp
from jax.experimental import pallas as pl
from jax.experimental.pallas import tpu as pltpu

_C = 64
_D = 64
_TEMP = 10.0

_MM = (((1,), (0,)), ((), ()))
_MM_T = (((0,), (0,)), ((), ()))


def _row_form(v):
  r = jax.lax.broadcasted_iota(jnp.int32, (_C, _C), 0)
  c = jax.lax.broadcasted_iota(jnp.int32, (_C, _C), 1)
  eye = (r == c).astype(jnp.float32)
  return jnp.sum(eye * v, axis=0, keepdims=True)


def _loss_kernel(lab1_ref, lab2_ref, f1_ref, f2_ref, out_ref,
                 sum1_ref, sum2_ref, cnt1_ref, cnt2_ref,
                 c1_ref, c2_ref, dsum1_ref, dsum2_ref, *, nblocks, block):
  ph = pl.program_id(0)
  b = pl.program_id(1)

  lab1 = lab1_ref[0]
  lab2 = lab2_ref[0]
  f1 = f1_ref[...]
  f2 = f2_ref[...]

  iota = jax.lax.broadcasted_iota(jnp.int32, (_C, block), 0)
  oh1 = (lab1 == iota).astype(jnp.float32)
  oh2 = (lab2 == iota).astype(jnp.float32)

  @pl.when(jnp.logical_and(ph == 0, b == 0))
  def _init():
    sum1_ref[...] = jnp.zeros_like(sum1_ref)
    sum2_ref[...] = jnp.zeros_like(sum2_ref)
    cnt1_ref[...] = jnp.zeros_like(cnt1_ref)
    cnt2_ref[...] = jnp.zeros_like(cnt2_ref)

  @pl.when(ph == 0)
  def _phase0():
    sum1_ref[...] += jax.lax.dot_general(
        oh1, f1, _MM, preferred_element_type=jnp.float32)
    sum2_ref[...] += jax.lax.dot_general(
        oh2, f2, _MM, preferred_element_type=jnp.float32)
    cnt1_ref[...] += jnp.sum(oh1, axis=1, keepdims=True)
    cnt2_ref[...] += jnp.sum(oh2, axis=1, keepdims=True)

  @pl.when(jnp.logical_and(ph == 1, b == 0))
  def _centers():
    safe1 = jnp.maximum(cnt1_ref[...], 1.0)
    safe2 = jnp.maximum(cnt2_ref[...], 1.0)
    c1_ref[...] = sum1_ref[...] / safe1
    c2_ref[...] = sum2_ref[...] / safe2
    dsum1_ref[...] = jnp.zeros_like(dsum1_ref)
    dsum2_ref[...] = jnp.zeros_like(dsum2_ref)

  @pl.when(ph == 1)
  def _phase1():
    cb1 = jax.lax.dot_general(
        oh1, c1_ref[...], _MM_T, preferred_element_type=jnp.float32)
    cb2 = jax.lax.dot_general(
        oh2, c2_ref[...], _MM_T, preferred_element_type=jnp.float32)
    diff1 = f1 - cb1
    diff2 = f2 - cb2
    d1 = jnp.sqrt(jnp.maximum(
        jnp.sum(diff1 * diff1, axis=1, keepdims=True), 1e-24))
    d2 = jnp.sqrt(jnp.maximum(
        jnp.sum(diff2 * diff2, axis=1, keepdims=True), 1e-24))
    dsum1_ref[...] += jax.lax.dot_general(
        oh1, d1, _MM, preferred_element_type=jnp.float32)
    dsum2_ref[...] += jax.lax.dot_general(
        oh2, d2, _MM, preferred_element_type=jnp.float32)

  @pl.when(jnp.logical_and(ph == 1, b == nblocks - 1))
  def _final():
    cnt1 = cnt1_ref[...]
    cnt2 = cnt2_ref[...]
    safe1 = jnp.maximum(cnt1, 1.0)
    safe2 = jnp.maximum(cnt2, 1.0)
    mean_d1 = dsum1_ref[...] / safe1
    mean_d2 = dsum2_ref[...] / safe2
    mask_intra = jnp.logical_and(cnt1 > 1.0, cnt2 > 1.0)
    intra = jnp.sum(jnp.where(mask_intra, mean_d1 + mean_d2, 0.0),
                    keepdims=True)

    c1 = c1_ref[...]
    c2 = c2_ref[...]
    n1 = jnp.sum(c1 * c1, axis=1, keepdims=True)
    n2 = jnp.sum(c2 * c2, axis=1, keepdims=True)
    gram = jax.lax.dot_general(
        c1, c2, (((1,), (1,)), ((), ())),
        preferred_element_type=jnp.float32)
    dsq = n1 + _row_form(n2) - 2.0 * gram
    dmat = jnp.sqrt(jnp.maximum(dsq, 1e-24))

    mask = jnp.logical_and(cnt1 > 0.0, cnt2 > 0.0).astype(jnp.float32)
    n_valid = jnp.sum(mask, keepdims=True)
    rowsum = jnp.sum(dmat * mask, axis=0, keepdims=True)
    masked_total = jax.lax.dot_general(
        rowsum, mask, (((1,), (0,)), ((), ())),
        preferred_element_type=jnp.float32)
    pair_cnt = jnp.maximum(n_valid * n_valid, 1.0)
    inter = jnp.where(n_valid > 1.0, masked_total / pair_cnt, 0.0)

    normalized = intra / (inter + 1e-8)
    loss = jnp.where(inter > 0.0,
                     jnp.log(1.0 + jnp.exp(normalized / _TEMP)),
                     intra)
    out_ref[...] = loss


@functools.partial(jax.jit, static_argnames=("block",))
def _dist_loss_pallas(feat1, label1, feat2, label2, block=2048):
  n, d = feat1.shape
  nblocks = n // block
  lab1 = label1.astype(jnp.int32).reshape(nblocks, 1, block)
  lab2 = label2.astype(jnp.int32).reshape(nblocks, 1, block)

  out = pl.pallas_call(
      functools.partial(_loss_kernel, nblocks=nblocks, block=block),
      grid=(2, nblocks),
      in_specs=[
          pl.BlockSpec((1, 1, block), lambda ph, b: (b, 0, 0)),
          pl.BlockSpec((1, 1, block), lambda ph, b: (b, 0, 0)),
          pl.BlockSpec((block, d), lambda ph, b: (b, 0)),
          pl.BlockSpec((block, d), lambda ph, b: (b, 0)),
      ],
      out_specs=pl.BlockSpec((1, 1), lambda ph, b: (0, 0)),
      out_shape=jax.ShapeDtypeStruct((1, 1), jnp.float32),
      scratch_shapes=[
          pltpu.VMEM((_C, _D), jnp.float32),
          pltpu.VMEM((_C, _D), jnp.float32),
          pltpu.VMEM((_C, 1), jnp.float32),
          pltpu.VMEM((_C, 1), jnp.float32),
          pltpu.VMEM((_C, _D), jnp.float32),
          pltpu.VMEM((_C, _D), jnp.float32),
          pltpu.VMEM((_C, 1), jnp.float32),
          pltpu.VMEM((_C, 1), jnp.float32),
      ],
  )(lab1, lab2, feat1, feat2)
  return out[0, 0]


def kernel(feat1, label1, feat2, label2):
  return _dist_loss_pallas(feat1, label1, feat2, label2)


# speedup vs baseline: 8.4571x; 1.1646x over previous
import functools

import jax
import jax.numpy as jnp
from jax.experimental import pallas as pl
from jax.experimental.pallas import tpu as pltpu

_C = 64
_D = 64
_TEMP = 10.0

_MM = (((1,), (0,)), ((), ()))
_MM_T = (((0,), (0,)), ((), ()))


def _row_form(v):
  r = jax.lax.broadcasted_iota(jnp.int32, (_C, _C), 0)
  c = jax.lax.broadcasted_iota(jnp.int32, (_C, _C), 1)
  eye = (r == c).astype(jnp.float32)
  return jnp.sum(eye * v, axis=0, keepdims=True)


def _loss_kernel(lab1_ref, lab2_ref, f1_ref, f2_ref, out_ref,
                 sum1_ref, sum2_ref, cnt1_ref, cnt2_ref,
                 c1_ref, c2_ref, dsum1_ref, dsum2_ref, *, nblocks, block):
  ph = pl.program_id(0)
  b = pl.program_id(1)

  lab1 = lab1_ref[0]
  lab2 = lab2_ref[0]
  f1 = f1_ref[...]
  f2 = f2_ref[...]

  iota = jax.lax.broadcasted_iota(jnp.int32, (_C, block), 0)
  oh1 = (lab1 == iota).astype(jnp.float32)
  oh2 = (lab2 == iota).astype(jnp.float32)

  @pl.when(jnp.logical_and(ph == 0, b == 0))
  def _init():
    sum1_ref[...] = jnp.zeros_like(sum1_ref)
    sum2_ref[...] = jnp.zeros_like(sum2_ref)
    cnt1_ref[...] = jnp.zeros_like(cnt1_ref)
    cnt2_ref[...] = jnp.zeros_like(cnt2_ref)

  @pl.when(ph == 0)
  def _phase0():
    sum1_ref[...] += jax.lax.dot_general(
        oh1, f1, _MM, preferred_element_type=jnp.float32)
    sum2_ref[...] += jax.lax.dot_general(
        oh2, f2, _MM, preferred_element_type=jnp.float32)
    cnt1_ref[...] += jnp.sum(oh1, axis=1, keepdims=True)
    cnt2_ref[...] += jnp.sum(oh2, axis=1, keepdims=True)

  @pl.when(jnp.logical_and(ph == 1, b == 0))
  def _centers():
    safe1 = jnp.maximum(cnt1_ref[...], 1.0)
    safe2 = jnp.maximum(cnt2_ref[...], 1.0)
    c1_ref[...] = sum1_ref[...] / safe1
    c2_ref[...] = sum2_ref[...] / safe2
    dsum1_ref[...] = jnp.zeros_like(dsum1_ref)
    dsum2_ref[...] = jnp.zeros_like(dsum2_ref)

  @pl.when(ph == 1)
  def _phase1():
    cb1 = jax.lax.dot_general(
        oh1, c1_ref[...], _MM_T, preferred_element_type=jnp.float32)
    cb2 = jax.lax.dot_general(
        oh2, c2_ref[...], _MM_T, preferred_element_type=jnp.float32)
    diff1 = f1 - cb1
    diff2 = f2 - cb2
    d1 = jnp.sqrt(jnp.maximum(
        jnp.sum(diff1 * diff1, axis=1, keepdims=True), 1e-24))
    d2 = jnp.sqrt(jnp.maximum(
        jnp.sum(diff2 * diff2, axis=1, keepdims=True), 1e-24))
    dsum1_ref[...] += jax.lax.dot_general(
        oh1, d1, _MM, preferred_element_type=jnp.float32)
    dsum2_ref[...] += jax.lax.dot_general(
        oh2, d2, _MM, preferred_element_type=jnp.float32)

  @pl.when(jnp.logical_and(ph == 1, b == nblocks - 1))
  def _final():
    cnt1 = cnt1_ref[...]
    cnt2 = cnt2_ref[...]
    safe1 = jnp.maximum(cnt1, 1.0)
    safe2 = jnp.maximum(cnt2, 1.0)
    mean_d1 = dsum1_ref[...] / safe1
    mean_d2 = dsum2_ref[...] / safe2
    mask_intra = jnp.logical_and(cnt1 > 1.0, cnt2 > 1.0)
    intra = jnp.sum(jnp.where(mask_intra, mean_d1 + mean_d2, 0.0),
                    keepdims=True)

    c1 = c1_ref[...]
    c2 = c2_ref[...]
    n1 = jnp.sum(c1 * c1, axis=1, keepdims=True)
    n2 = jnp.sum(c2 * c2, axis=1, keepdims=True)
    gram = jax.lax.dot_general(
        c1, c2, (((1,), (1,)), ((), ())),
        preferred_element_type=jnp.float32)
    dsq = n1 + _row_form(n2) - 2.0 * gram
    dmat = jnp.sqrt(jnp.maximum(dsq, 1e-24))

    mask = jnp.logical_and(cnt1 > 0.0, cnt2 > 0.0).astype(jnp.float32)
    n_valid = jnp.sum(mask, keepdims=True)
    rowsum = jnp.sum(dmat * mask, axis=0, keepdims=True)
    masked_total = jax.lax.dot_general(
        rowsum, mask, (((1,), (0,)), ((), ())),
        preferred_element_type=jnp.float32)
    pair_cnt = jnp.maximum(n_valid * n_valid, 1.0)
    inter = jnp.where(n_valid > 1.0, masked_total / pair_cnt, 0.0)

    normalized = intra / (inter + 1e-8)
    loss = jnp.where(inter > 0.0,
                     jnp.log(1.0 + jnp.exp(normalized / _TEMP)),
                     intra)
    out_ref[...] = loss


@functools.partial(jax.jit, static_argnames=("block",))
def _dist_loss_pallas(feat1, label1, feat2, label2, block=8192):
  n, d = feat1.shape
  nblocks = n // block
  lab1 = label1.astype(jnp.int32).reshape(nblocks, 1, block)
  lab2 = label2.astype(jnp.int32).reshape(nblocks, 1, block)

  out = pl.pallas_call(
      functools.partial(_loss_kernel, nblocks=nblocks, block=block),
      grid=(2, nblocks),
      in_specs=[
          pl.BlockSpec((1, 1, block), lambda ph, b: (b, 0, 0)),
          pl.BlockSpec((1, 1, block), lambda ph, b: (b, 0, 0)),
          pl.BlockSpec((block, d), lambda ph, b: (b, 0)),
          pl.BlockSpec((block, d), lambda ph, b: (b, 0)),
      ],
      out_specs=pl.BlockSpec((1, 1), lambda ph, b: (0, 0)),
      out_shape=jax.ShapeDtypeStruct((1, 1), jnp.float32),
      scratch_shapes=[
          pltpu.VMEM((_C, _D), jnp.float32),
          pltpu.VMEM((_C, _D), jnp.float32),
          pltpu.VMEM((_C, 1), jnp.float32),
          pltpu.VMEM((_C, 1), jnp.float32),
          pltpu.VMEM((_C, _D), jnp.float32),
          pltpu.VMEM((_C, _D), jnp.float32),
          pltpu.VMEM((_C, 1), jnp.float32),
          pltpu.VMEM((_C, 1), jnp.float32),
      ],
  )(lab1, lab2, feat1, feat2)
  return out[0, 0]


def kernel(feat1, label1, feat2, label2):
  return _dist_loss_pallas(feat1, label1, feat2, label2)
